# baseline (speedup 1.0000x reference)
import jax
import jax.numpy as jnp
from jax import lax
from jax.experimental import pallas as pl
from jax.experimental.pallas import tpu as pltpu
from jax.experimental.pallas import tpu_sc as plsc

NC = 2
NS = 16
NW = NC * NS
L = 16

VOCAB = 1000000
DIM = 64
BATCH = 16384
NUM_CTX = 5
WIDE = 2 * DIM

CHUNK = 64
ROWS_PER_W = BATCH // NW
N_CHUNKS = ROWS_PER_W // CHUNK

GRP = 16
N_GRP = CHUNK // GRP
PAIRS = GRP * NUM_CTX

TBLK = 32768
HB = TBLK // 2
TGRID = -(-VOCAB // TBLK)
NSUPER = TGRID * HB


def _tc_transpose_kernel(x_ref, o_ref):
  eye = jnp.eye(DIM, dtype=jnp.float32)
  t = lax.dot_general(
      x_ref[...], eye, (((0,), (0,)), ((), ())),
      preferred_element_type=jnp.float32)
  o_ref[:, 0:DIM] = t[0:HB, :]
  o_ref[:, DIM:WIDE] = t[HB:TBLK, :]


def _to_super_rows(table_t):
  return pl.pallas_call(
      _tc_transpose_kernel,
      grid=(TGRID,),
      in_specs=[pl.BlockSpec((DIM, TBLK), lambda g: (0, g))],
      out_specs=pl.BlockSpec((HB, WIDE), lambda g: (g, 0)),
      out_shape=jax.ShapeDtypeStruct((NSUPER, WIDE), jnp.float32),
  )(table_t)


def _split_idx(v):
  i = v % TBLK
  sup = (v // TBLK) * HB + (i % HB)
  half = (i // HB) * DIM
  return sup, half


def _sc_kernel(sg_wide, ctx_wide, tgt_sup, tgt_half, ctx_sup, ctx_half, out,
               tgt_idx, tgt_hlf, ctx_idx, ctx_hlf, tgt_rows, ctx_rows,
               prods, out_v, sem):
  wid = lax.axis_index("s") * NC + lax.axis_index("c")
  iota = lax.iota(jnp.int32, L)
  gather_base = iota * L

  def stage_and_fire(c, nb):
    base = wid * ROWS_PER_W + c * CHUNK
    pltpu.sync_copy(tgt_sup.at[pl.ds(base, CHUNK)],
                    tgt_idx.at[pl.ds(nb * CHUNK, CHUNK)])
    pltpu.sync_copy(tgt_half.at[pl.ds(base, CHUNK)],
                    tgt_hlf.at[pl.ds(nb * CHUNK, CHUNK)])
    for j in range(NUM_CTX):
      pltpu.sync_copy(
          ctx_sup.at[pl.ds(j * BATCH + base, CHUNK)],
          ctx_idx.at[pl.ds((nb * NUM_CTX + j) * CHUNK, CHUNK)])
    pltpu.sync_copy(ctx_half.at[pl.ds(base * NUM_CTX, CHUNK * NUM_CTX)],
                    ctx_hlf.at[pl.ds(nb * NUM_CTX * CHUNK,
                                     NUM_CTX * CHUNK)])
    pltpu.async_copy(
        sg_wide.at[tgt_idx.at[pl.ds(nb * CHUNK, CHUNK)]],
        tgt_rows.at[pl.ds(nb * CHUNK, CHUNK)], sem)
    for j in range(NUM_CTX):
      pltpu.async_copy(
          ctx_wide.at[ctx_idx.at[pl.ds((nb * NUM_CTX + j) * CHUNK, CHUNK)]],
          ctx_rows.at[pl.ds((nb * NUM_CTX + j) * CHUNK, CHUNK)], sem)

  def wait_chunk(nb):
    pltpu.make_async_copy(
        sg_wide.at[tgt_idx.at[pl.ds(nb * CHUNK, CHUNK)]],
        tgt_rows.at[pl.ds(nb * CHUNK, CHUNK)], sem).wait()
    for j in range(NUM_CTX):
      pltpu.make_async_copy(
          ctx_wide.at[ctx_idx.at[pl.ds((nb * NUM_CTX + j) * CHUNK, CHUNK)]],
          ctx_rows.at[pl.ds((nb * NUM_CTX + j) * CHUNK, CHUNK)], sem).wait()

  def compute(c, nb):
    base = wid * ROWS_PER_W + c * CHUNK

    def body(g, carry):
      b0 = g * GRP
      thv = tgt_hlf[pl.ds(nb * CHUNK + b0, L)]
      chv = [ctx_hlf[pl.ds(nb * NUM_CTX * CHUNK + b0 * NUM_CTX + q * L, L)]
             for q in range(PAIRS // L)]
      for bi in range(GRP):
        b = b0 + bi
        tho = thv[bi]
        brow = jnp.full((L,), nb * CHUNK + b, jnp.int32)
        tb = [plsc.load_gather(tgt_rows, [brow, iota + (tho + k * L)])
              for k in range(DIM // L)]
        for j in range(NUM_CTX):
          p = bi * NUM_CTX + j
          cho = chv[p // L][p % L]
          rrow = jnp.full((L,), (nb * NUM_CTX + j) * CHUNK + b, jnp.int32)
          ccols = iota + cho
          acc = plsc.load_gather(ctx_rows, [rrow, ccols]) * tb[0]
          for k in range(1, DIM // L):
            acc = acc + plsc.load_gather(
                ctx_rows, [rrow, ccols + k * L]) * tb[k]
          prods[pl.ds(p * L, L)] = acc
      for o in range(PAIRS // L):
        sums = plsc.load_gather(prods, [gather_base + o * (L * L)])
        for k in range(1, L):
          sums = sums + plsc.load_gather(
              prods, [gather_base + (o * (L * L) + k)])
        out_v[pl.ds(b0 * NUM_CTX + o * L, L)] = sums
      return carry

    lax.fori_loop(0, N_GRP, body, 0)
    pltpu.sync_copy(out_v, out.at[pl.ds(base * NUM_CTX, CHUNK * NUM_CTX)])

  stage_and_fire(0, 0)

  def chunk_body(c, carry):
    nb = c % 2
    wait_chunk(nb)

    @pl.when(c + 1 < N_CHUNKS)
    def _():
      stage_and_fire(c + 1, 1 - nb)

    compute(c, nb)
    return carry

  lax.fori_loop(0, N_CHUNKS, chunk_body, 0)


@jax.jit
def _run(tgt_sup, tgt_half, ctx_sup, ctx_half, sg_t, ctx_t):
  sg_wide = _to_super_rows(sg_t)
  ctx_wide = _to_super_rows(ctx_t)
  mesh = plsc.VectorSubcoreMesh(core_axis_name="c", subcore_axis_name="s")
  return pl.kernel(
      _sc_kernel,
      out_type=jax.ShapeDtypeStruct((BATCH * NUM_CTX,), jnp.float32),
      mesh=mesh,
      compiler_params=pltpu.CompilerParams(needs_layout_passes=False),
      scratch_types=[
          pltpu.VMEM((2 * CHUNK,), jnp.int32),
          pltpu.VMEM((2 * CHUNK,), jnp.int32),
          pltpu.VMEM((2 * NUM_CTX * CHUNK,), jnp.int32),
          pltpu.VMEM((2 * NUM_CTX * CHUNK,), jnp.int32),
          pltpu.VMEM((2 * CHUNK, WIDE), jnp.float32),
          pltpu.VMEM((2 * NUM_CTX * CHUNK, WIDE), jnp.float32),
          pltpu.VMEM((PAIRS * L,), jnp.float32),
          pltpu.VMEM((CHUNK * NUM_CTX,), jnp.float32),
          pltpu.SemaphoreType.DMA,
      ],
  )(sg_wide, ctx_wide, tgt_sup, tgt_half, ctx_sup, ctx_half)


def kernel(target, context, sg_table, ctx_table):
  target = target.astype(jnp.int32)
  context = context.astype(jnp.int32)
  tgt_sup, tgt_half = _split_idx(target)
  csup, chalf = _split_idx(context)
  ctx_sup = jnp.transpose(csup, (1, 0)).reshape(-1)
  ctx_half = chalf.reshape(-1)
  out_flat = _run(tgt_sup, tgt_half, ctx_sup, ctx_half,
                  sg_table.T, ctx_table.T)
  return out_flat.reshape(BATCH, NUM_CTX)


# speedup vs baseline: 1.0145x; 1.0145x over previous
import jax
import jax.numpy as jnp
from jax import lax
from jax.experimental import pallas as pl
from jax.experimental.pallas import tpu as pltpu
from jax.experimental.pallas import tpu_sc as plsc

NC = 2
NS = 16
NW = NC * NS
L = 16

VOCAB = 1000000
DIM = 64
BATCH = 16384
NUM_CTX = 5
WIDE = 2 * DIM

CHUNK = 64
ROWS_PER_W = BATCH // NW
N_CHUNKS = ROWS_PER_W // CHUNK

GRP = 16
N_GRP = CHUNK // GRP
PAIRS = GRP * NUM_CTX

TBLK = 32768
HB = TBLK // 2
TGRID = -(-VOCAB // TBLK)
NSUPER = TGRID * HB


def _tc_transpose_kernel(x_ref, o_ref):
  eye = jnp.eye(DIM, dtype=jnp.float32)
  t = lax.dot_general(
      x_ref[...], eye, (((0,), (0,)), ((), ())),
      preferred_element_type=jnp.float32)
  o_ref[:, 0:DIM] = t[0:HB, :]
  o_ref[:, DIM:WIDE] = t[HB:TBLK, :]


def _to_super_rows(table_t):
  return pl.pallas_call(
      _tc_transpose_kernel,
      grid=(TGRID,),
      in_specs=[pl.BlockSpec((DIM, TBLK), lambda g: (0, g))],
      out_specs=pl.BlockSpec((HB, WIDE), lambda g: (g, 0)),
      out_shape=jax.ShapeDtypeStruct((NSUPER, WIDE), jnp.float32),
  )(table_t)


def _split_idx(v):
  i = v % TBLK
  sup = (v // TBLK) * HB + (i % HB)
  half = (i // HB) * DIM
  return sup, half


def _sc_tgt_kernel(sg_wide, tgt_sup, tgt_half, out,
                   idx_v, hlf_v, rows_v, out_v, sem):
  wid = lax.axis_index("s") * NC + lax.axis_index("c")
  iota = lax.iota(jnp.int32, L)
  base = wid * ROWS_PER_W

  pltpu.sync_copy(tgt_sup.at[pl.ds(base, ROWS_PER_W)], idx_v)
  pltpu.sync_copy(tgt_half.at[pl.ds(base, ROWS_PER_W)], hlf_v)
  copies = []
  for q in range(ROWS_PER_W // 128):
    copies.append(pltpu.async_copy(
        sg_wide.at[idx_v.at[pl.ds(q * 128, 128)]],
        rows_v.at[pl.ds(q * 128, 128)], sem))
  for cp in copies:
    cp.wait()

  def body(g, carry):
    b0 = g * GRP
    thv = hlf_v[pl.ds(b0, L)]
    for bi in range(GRP):
      b = b0 + bi
      tho = thv[bi]
      brow = jnp.full((L,), b, jnp.int32)
      for k in range(DIM // L):
        out_v[pl.ds(b * DIM + k * L, L)] = plsc.load_gather(
            rows_v, [brow, iota + (tho + k * L)])
    return carry

  lax.fori_loop(0, ROWS_PER_W // GRP, body, 0)
  pltpu.sync_copy(out_v, out.at[pl.ds(base * DIM, ROWS_PER_W * DIM)])


def _sc_kernel(ctx_wide, tgtvec, ctx_sup, ctx_half, out,
               ctx_idx, ctx_hlf, tgt_flat, ctx_rows, prods, out_v, sem):
  wid = lax.axis_index("s") * NC + lax.axis_index("c")
  iota = lax.iota(jnp.int32, L)
  gather_base = iota * L

  def stage_and_fire(c, nb):
    base = wid * ROWS_PER_W + c * CHUNK
    for j in range(NUM_CTX):
      pltpu.sync_copy(
          ctx_sup.at[pl.ds(j * BATCH + base, CHUNK)],
          ctx_idx.at[pl.ds((nb * NUM_CTX + j) * CHUNK, CHUNK)])
    pltpu.sync_copy(ctx_half.at[pl.ds(base * NUM_CTX, CHUNK * NUM_CTX)],
                    ctx_hlf.at[pl.ds(nb * NUM_CTX * CHUNK,
                                     NUM_CTX * CHUNK)])
    pltpu.async_copy(
        tgtvec.at[pl.ds(base * DIM, CHUNK * DIM)],
        tgt_flat.at[pl.ds(nb * CHUNK * DIM, CHUNK * DIM)], sem)
    for j in range(NUM_CTX):
      pltpu.async_copy(
          ctx_wide.at[ctx_idx.at[pl.ds((nb * NUM_CTX + j) * CHUNK, CHUNK)]],
          ctx_rows.at[pl.ds((nb * NUM_CTX + j) * CHUNK, CHUNK)], sem)

  def wait_chunk(nb):
    pltpu.make_async_copy(
        tgtvec.at[pl.ds(0, CHUNK * DIM)],
        tgt_flat.at[pl.ds(nb * CHUNK * DIM, CHUNK * DIM)], sem).wait()
    for j in range(NUM_CTX):
      pltpu.make_async_copy(
          ctx_wide.at[ctx_idx.at[pl.ds((nb * NUM_CTX + j) * CHUNK, CHUNK)]],
          ctx_rows.at[pl.ds((nb * NUM_CTX + j) * CHUNK, CHUNK)], sem).wait()

  def compute(c, nb):
    base = wid * ROWS_PER_W + c * CHUNK

    def body(g, carry):
      b0 = g * GRP
      chv = [ctx_hlf[pl.ds(nb * NUM_CTX * CHUNK + b0 * NUM_CTX + q * L, L)]
             for q in range(PAIRS // L)]
      for bi in range(GRP):
        b = b0 + bi
        toff = (nb * CHUNK + b) * DIM
        tb = [tgt_flat[pl.ds(toff + k * L, L)] for k in range(DIM // L)]
        for j in range(NUM_CTX):
          p = bi * NUM_CTX + j
          cho = chv[p // L][p % L]
          rrow = jnp.full((L,), (nb * NUM_CTX + j) * CHUNK + b, jnp.int32)
          ccols = iota + cho
          acc = plsc.load_gather(ctx_rows, [rrow, ccols]) * tb[0]
          for k in range(1, DIM // L):
            acc = acc + plsc.load_gather(
                ctx_rows, [rrow, ccols + k * L]) * tb[k]
          prods[pl.ds(p * L, L)] = acc
      for o in range(PAIRS // L):
        sums = plsc.load_gather(prods, [gather_base + o * (L * L)])
        for k in range(1, L):
          sums = sums + plsc.load_gather(
              prods, [gather_base + (o * (L * L) + k)])
        out_v[pl.ds(b0 * NUM_CTX + o * L, L)] = sums
      return carry

    lax.fori_loop(0, N_GRP, body, 0)
    pltpu.sync_copy(out_v, out.at[pl.ds(base * NUM_CTX, CHUNK * NUM_CTX)])

  stage_and_fire(0, 0)

  def chunk_body(c, carry):
    nb = c % 2
    wait_chunk(nb)

    @pl.when(c + 1 < N_CHUNKS)
    def _():
      stage_and_fire(c + 1, 1 - nb)

    compute(c, nb)
    return carry

  lax.fori_loop(0, N_CHUNKS, chunk_body, 0)


@jax.jit
def _run(tgt_sup, tgt_half, ctx_sup, ctx_half, sg_t, ctx_t):
  mesh = plsc.VectorSubcoreMesh(core_axis_name="c", subcore_axis_name="s")
  params = pltpu.CompilerParams(needs_layout_passes=False)

  sg_wide = _to_super_rows(sg_t)
  tgtvec = pl.kernel(
      _sc_tgt_kernel,
      out_type=jax.ShapeDtypeStruct((BATCH * DIM,), jnp.float32),
      mesh=mesh,
      compiler_params=params,
      scratch_types=[
          pltpu.VMEM((ROWS_PER_W,), jnp.int32),
          pltpu.VMEM((ROWS_PER_W,), jnp.int32),
          pltpu.VMEM((ROWS_PER_W, WIDE), jnp.float32),
          pltpu.VMEM((ROWS_PER_W * DIM,), jnp.float32),
          pltpu.SemaphoreType.DMA,
      ],
  )(sg_wide, tgt_sup, tgt_half)

  ctx_wide = _to_super_rows(ctx_t)
  return pl.kernel(
      _sc_kernel,
      out_type=jax.ShapeDtypeStruct((BATCH * NUM_CTX,), jnp.float32),
      mesh=mesh,
      compiler_params=params,
      scratch_types=[
          pltpu.VMEM((2 * NUM_CTX * CHUNK,), jnp.int32),
          pltpu.VMEM((2 * NUM_CTX * CHUNK,), jnp.int32),
          pltpu.VMEM((2 * CHUNK * DIM,), jnp.float32),
          pltpu.VMEM((2 * NUM_CTX * CHUNK, WIDE), jnp.float32),
          pltpu.VMEM((PAIRS * L,), jnp.float32),
          pltpu.VMEM((CHUNK * NUM_CTX,), jnp.float32),
          pltpu.SemaphoreType.DMA,
      ],
  )(ctx_wide, tgtvec, ctx_sup, ctx_half)


def kernel(target, context, sg_table, ctx_table):
  target = target.astype(jnp.int32)
  context = context.astype(jnp.int32)
  tgt_sup, tgt_half = _split_idx(target)
  csup, chalf = _split_idx(context)
  ctx_sup = jnp.transpose(csup, (1, 0)).reshape(-1)
  ctx_half = chalf.reshape(-1)
  out_flat = _run(tgt_sup, tgt_half, ctx_sup, ctx_half,
                  sg_table.T, ctx_table.T)
  return out_flat.reshape(BATCH, NUM_CTX)


# speedup vs baseline: 1.0169x; 1.0024x over previous
import jax
import jax.numpy as jnp
from jax import lax
from jax.experimental import pallas as pl
from jax.experimental.pallas import tpu as pltpu
from jax.experimental.pallas import tpu_sc as plsc

NC = 2
NS = 16
NW = NC * NS
L = 16

VOCAB = 1000000
DIM = 64
BATCH = 16384
NUM_CTX = 5
WIDE = 2 * DIM

CHUNK = 64
ROWS_PER_W = BATCH // NW
N_CHUNKS = ROWS_PER_W // CHUNK

GRP = 16
N_GRP = CHUNK // GRP
PAIRS = GRP * NUM_CTX

TBLK = 32768
HB = TBLK // 2
TGRID = -(-VOCAB // TBLK)
NSUPER = TGRID * HB


def _tc_transpose_kernel(x_ref, o_ref):
  t = x_ref[...].T
  o_ref[:, 0:DIM] = t[0:HB, :]
  o_ref[:, DIM:WIDE] = t[HB:TBLK, :]


def _to_super_rows(table_t):
  return pl.pallas_call(
      _tc_transpose_kernel,
      grid=(TGRID,),
      in_specs=[pl.BlockSpec((DIM, TBLK), lambda g: (0, g))],
      out_specs=pl.BlockSpec((HB, WIDE), lambda g: (g, 0)),
      out_shape=jax.ShapeDtypeStruct((NSUPER, WIDE), jnp.float32),
  )(table_t)


def _split_idx(v):
  i = v % TBLK
  sup = (v // TBLK) * HB + (i % HB)
  half = (i // HB) * DIM
  return sup, half


def _sc_tgt_kernel(sg_wide, tgt_sup, tgt_half, out,
                   idx_v, hlf_v, rows_v, out_v, sem):
  wid = lax.axis_index("s") * NC + lax.axis_index("c")
  iota = lax.iota(jnp.int32, L)
  base = wid * ROWS_PER_W

  pltpu.sync_copy(tgt_sup.at[pl.ds(base, ROWS_PER_W)], idx_v)
  pltpu.sync_copy(tgt_half.at[pl.ds(base, ROWS_PER_W)], hlf_v)
  copies = []
  for q in range(ROWS_PER_W // 128):
    copies.append(pltpu.async_copy(
        sg_wide.at[idx_v.at[pl.ds(q * 128, 128)]],
        rows_v.at[pl.ds(q * 128, 128)], sem))
  for cp in copies:
    cp.wait()

  def body(g, carry):
    b0 = g * GRP
    thv = hlf_v[pl.ds(b0, L)]
    for bi in range(GRP):
      b = b0 + bi
      tho = thv[bi]
      brow = jnp.full((L,), b, jnp.int32)
      for k in range(DIM // L):
        out_v[pl.ds(b * DIM + k * L, L)] = plsc.load_gather(
            rows_v, [brow, iota + (tho + k * L)])
    return carry

  lax.fori_loop(0, ROWS_PER_W // GRP, body, 0)
  pltpu.sync_copy(out_v, out.at[pl.ds(base * DIM, ROWS_PER_W * DIM)])


def _sc_kernel(ctx_wide, tgtvec, ctx_sup, ctx_half, out,
               ctx_idx, ctx_hlf, tgt_flat, ctx_rows, prods, out_v, sem):
  wid = lax.axis_index("s") * NC + lax.axis_index("c")
  iota = lax.iota(jnp.int32, L)
  gather_base = iota * L

  def stage_and_fire(c, nb):
    base = wid * ROWS_PER_W + c * CHUNK
    for j in range(NUM_CTX):
      pltpu.sync_copy(
          ctx_sup.at[pl.ds(j * BATCH + base, CHUNK)],
          ctx_idx.at[pl.ds((nb * NUM_CTX + j) * CHUNK, CHUNK)])
    pltpu.sync_copy(ctx_half.at[pl.ds(base * NUM_CTX, CHUNK * NUM_CTX)],
                    ctx_hlf.at[pl.ds(nb * NUM_CTX * CHUNK,
                                     NUM_CTX * CHUNK)])
    pltpu.async_copy(
        tgtvec.at[pl.ds(base * DIM, CHUNK * DIM)],
        tgt_flat.at[pl.ds(nb * CHUNK * DIM, CHUNK * DIM)], sem)
    for j in range(NUM_CTX):
      pltpu.async_copy(
          ctx_wide.at[ctx_idx.at[pl.ds((nb * NUM_CTX + j) * CHUNK, CHUNK)]],
          ctx_rows.at[pl.ds((nb * NUM_CTX + j) * CHUNK, CHUNK)], sem)

  def wait_chunk(nb):
    pltpu.make_async_copy(
        tgtvec.at[pl.ds(0, CHUNK * DIM)],
        tgt_flat.at[pl.ds(nb * CHUNK * DIM, CHUNK * DIM)], sem).wait()
    for j in range(NUM_CTX):
      pltpu.make_async_copy(
          ctx_wide.at[ctx_idx.at[pl.ds((nb * NUM_CTX + j) * CHUNK, CHUNK)]],
          ctx_rows.at[pl.ds((nb * NUM_CTX + j) * CHUNK, CHUNK)], sem).wait()

  def compute(c, nb):
    base = wid * ROWS_PER_W + c * CHUNK

    def body(g, carry):
      b0 = g * GRP
      chv = [ctx_hlf[pl.ds(nb * NUM_CTX * CHUNK + b0 * NUM_CTX + q * L, L)]
             for q in range(PAIRS // L)]
      for bi in range(GRP):
        b = b0 + bi
        toff = (nb * CHUNK + b) * DIM
        tb = [tgt_flat[pl.ds(toff + k * L, L)] for k in range(DIM // L)]
        for j in range(NUM_CTX):
          p = bi * NUM_CTX + j
          cho = chv[p // L][p % L]
          rrow = jnp.full((L,), (nb * NUM_CTX + j) * CHUNK + b, jnp.int32)
          ccols = iota + cho
          acc = plsc.load_gather(ctx_rows, [rrow, ccols]) * tb[0]
          for k in range(1, DIM // L):
            acc = acc + plsc.load_gather(
                ctx_rows, [rrow, ccols + k * L]) * tb[k]
          prods[pl.ds(p * L, L)] = acc
      for o in range(PAIRS // L):
        sums = plsc.load_gather(prods, [gather_base + o * (L * L)])
        for k in range(1, L):
          sums = sums + plsc.load_gather(
              prods, [gather_base + (o * (L * L) + k)])
        out_v[pl.ds(b0 * NUM_CTX + o * L, L)] = sums
      return carry

    lax.fori_loop(0, N_GRP, body, 0)
    pltpu.sync_copy(out_v, out.at[pl.ds(base * NUM_CTX, CHUNK * NUM_CTX)])

  stage_and_fire(0, 0)

  def chunk_body(c, carry):
    nb = c % 2
    wait_chunk(nb)

    @pl.when(c + 1 < N_CHUNKS)
    def _():
      stage_and_fire(c + 1, 1 - nb)

    compute(c, nb)
    return carry

  lax.fori_loop(0, N_CHUNKS, chunk_body, 0)


@jax.jit
def _run(tgt_sup, tgt_half, ctx_sup, ctx_half, sg_t, ctx_t):
  mesh = plsc.VectorSubcoreMesh(core_axis_name="c", subcore_axis_name="s")
  params = pltpu.CompilerParams(needs_layout_passes=False)

  sg_wide = _to_super_rows(sg_t)
  tgtvec = pl.kernel(
      _sc_tgt_kernel,
      out_type=jax.ShapeDtypeStruct((BATCH * DIM,), jnp.float32),
      mesh=mesh,
      compiler_params=params,
      scratch_types=[
          pltpu.VMEM((ROWS_PER_W,), jnp.int32),
          pltpu.VMEM((ROWS_PER_W,), jnp.int32),
          pltpu.VMEM((ROWS_PER_W, WIDE), jnp.float32),
          pltpu.VMEM((ROWS_PER_W * DIM,), jnp.float32),
          pltpu.SemaphoreType.DMA,
      ],
  )(sg_wide, tgt_sup, tgt_half)

  ctx_wide = _to_super_rows(ctx_t)
  return pl.kernel(
      _sc_kernel,
      out_type=jax.ShapeDtypeStruct((BATCH * NUM_CTX,), jnp.float32),
      mesh=mesh,
      compiler_params=params,
      scratch_types=[
          pltpu.VMEM((2 * NUM_CTX * CHUNK,), jnp.int32),
          pltpu.VMEM((2 * NUM_CTX * CHUNK,), jnp.int32),
          pltpu.VMEM((2 * CHUNK * DIM,), jnp.float32),
          pltpu.VMEM((2 * NUM_CTX * CHUNK, WIDE), jnp.float32),
          pltpu.VMEM((PAIRS * L,), jnp.float32),
          pltpu.VMEM((CHUNK * NUM_CTX,), jnp.float32),
          pltpu.SemaphoreType.DMA,
      ],
  )(ctx_wide, tgtvec, ctx_sup, ctx_half)


def kernel(target, context, sg_table, ctx_table):
  target = target.astype(jnp.int32)
  context = context.astype(jnp.int32)
  tgt_sup, tgt_half = _split_idx(target)
  csup, chalf = _split_idx(context)
  ctx_sup = jnp.transpose(csup, (1, 0)).reshape(-1)
  ctx_half = chalf.reshape(-1)
  out_flat = _run(tgt_sup, tgt_half, ctx_sup, ctx_half,
                  sg_table.T, ctx_table.T)
  return out_flat.reshape(BATCH, NUM_CTX)
